# baseline (speedup 1.0000x reference)
import functools

import jax
import jax.numpy as jnp
from jax import lax
from jax.experimental import pallas as pl
from jax.experimental.pallas import tpu as pltpu
from jax.experimental.pallas import tpu_sc as plsc

L = 16
NC = 2
NS = 16
NW = NC * NS
RG = 8
CHUNK_C = 2048
UNROLL = 16


@functools.cache
def _build_lookup(nrows: int, ncols: int, num_ops: int):
    n_row_groups = nrows // RG
    n_col_parts = NW // n_row_groups
    cols_per_tile = ncols // n_col_parts
    n_chunks = cols_per_tile // CHUNK_C
    n_pat_vregs = num_ops // L

    mesh = plsc.VectorSubcoreMesh(core_axis_name="c", subcore_axis_name="s")

    @functools.partial(
        pl.kernel,
        mesh=mesh,
        compiler_params=pltpu.CompilerParams(needs_layout_passes=False),
        out_type=jax.ShapeDtypeStruct((nrows, ncols), jnp.int32),
        scratch_types=[
            pltpu.VMEM((max(num_ops, 128),), jnp.int32),
            pltpu.VMEM((num_ops * L,), jnp.int32),
            pltpu.VMEM((RG, CHUNK_C), jnp.int32),
            pltpu.VMEM((RG, CHUNK_C), jnp.int32),
            pltpu.VMEM((RG, CHUNK_C), jnp.int32),
            pltpu.VMEM((RG, CHUNK_C), jnp.int32),
            pltpu.SemaphoreType.DMA,
            pltpu.SemaphoreType.DMA,
            pltpu.SemaphoreType.DMA,
            pltpu.SemaphoreType.DMA,
        ],
    )
    def lookup(t_hbm, pat_hbm, out_hbm, pat_v, lut_v,
               in0_v, in1_v, out0_v, out1_v, isem0, isem1, osem0, osem1):
        wid = lax.axis_index("s") * NC + lax.axis_index("c")
        row0 = pl.multiple_of((wid // n_col_parts) * RG, RG)
        col_base = pl.multiple_of((wid % n_col_parts) * cols_per_tile, CHUNK_C)
        in_bufs, out_bufs = (in0_v, in1_v), (out0_v, out1_v)
        in_sems, out_sems = (isem0, isem1), (osem0, osem1)

        def in_copy(ci):
            c0 = pl.multiple_of(col_base + ci * CHUNK_C, CHUNK_C)
            b = ci & 1
            return pltpu.make_async_copy(
                t_hbm.at[pl.ds(row0, RG), pl.ds(c0, CHUNK_C)],
                in_bufs[b], in_sems[b])

        def out_copy(ci):
            c0 = pl.multiple_of(col_base + ci * CHUNK_C, CHUNK_C)
            b = ci & 1
            return pltpu.make_async_copy(
                out_bufs[b],
                out_hbm.at[pl.ds(row0, RG), pl.ds(c0, CHUNK_C)], out_sems[b])

        in_copy(0).start()
        if n_chunks > 1:
            in_copy(1).start()

        pltpu.sync_copy(pat_hbm, pat_v.at[pl.ds(0, num_ops)])
        zeros = jnp.zeros((L,), jnp.int32)
        lane = lax.iota(jnp.int32, L)
        for v in range(num_ops):
            lut_v[pl.ds(v * L, L)] = zeros
        for j in reversed(range(n_pat_vregs)):
            pats = pat_v[pl.ds(j * L, L)]
            vals = lax.iota(jnp.int32, L) + (j * L)
            for l in range(L):
                plsc.store_scatter(lut_v, [pats * L + l], vals)

        for ci in range(n_chunks):
            b = ci & 1
            in_copy(ci).wait()
            if ci >= 2:
                out_copy(ci - 2).wait()

            for r in range(RG):
                def vec_body(g, c2, b=b, r=r):
                    for u in range(UNROLL):
                        o = pl.multiple_of(g * (L * UNROLL) + u * L, L)
                        idx = in_bufs[b][r, pl.ds(o, L)]
                        out_bufs[b][r, pl.ds(o, L)] = plsc.load_gather(
                            lut_v, [idx * L + lane])
                    return c2

                lax.fori_loop(0, CHUNK_C // (L * UNROLL), vec_body, 0)

            out_copy(ci).start()
            if ci + 2 < n_chunks:
                in_copy(ci + 2).start()

        for ci in range(max(0, n_chunks - 2), n_chunks):
            out_copy(ci).wait()

    return lookup


def kernel(tensor, op_patterns):
    nrows, ncols = tensor.shape
    lookup = _build_lookup(nrows, ncols, op_patterns.shape[0])
    return lookup(tensor, op_patterns)


# speedup vs baseline: 1.0461x; 1.0461x over previous
import functools

import jax
import jax.numpy as jnp
from jax import lax
from jax.experimental import pallas as pl
from jax.experimental.pallas import tpu as pltpu
from jax.experimental.pallas import tpu_sc as plsc

L = 16
NC = 2
NS = 16
NW = NC * NS
RG = 8
CHUNK_C = 2048
UNROLL = 16


@functools.cache
def _build_lookup(nrows: int, ncols: int, num_ops: int):
    n_row_groups = nrows // RG
    n_col_parts = NW // n_row_groups
    cols_per_tile = ncols // n_col_parts
    n_chunks = cols_per_tile // CHUNK_C
    n_pat_vregs = num_ops // L

    mesh = plsc.VectorSubcoreMesh(core_axis_name="c", subcore_axis_name="s")

    @functools.partial(
        pl.kernel,
        mesh=mesh,
        compiler_params=pltpu.CompilerParams(
            needs_layout_passes=False,
            disable_bounds_checks=True,
        ),
        out_type=jax.ShapeDtypeStruct((nrows, ncols), jnp.int32),
        scratch_types=[
            pltpu.VMEM((max(num_ops, 128),), jnp.int32),
            pltpu.VMEM((max(num_ops, 128),), jnp.int32),
            pltpu.VMEM((RG, CHUNK_C), jnp.int32),
            pltpu.VMEM((RG, CHUNK_C), jnp.int32),
            pltpu.VMEM((RG, CHUNK_C), jnp.int32),
            pltpu.VMEM((RG, CHUNK_C), jnp.int32),
            pltpu.SemaphoreType.DMA,
            pltpu.SemaphoreType.DMA,
            pltpu.SemaphoreType.DMA,
            pltpu.SemaphoreType.DMA,
        ],
    )
    def lookup(t_hbm, pat_hbm, out_hbm, pat_v, lut_v,
               in0_v, in1_v, out0_v, out1_v, isem0, isem1, osem0, osem1):
        wid = lax.axis_index("s") * NC + lax.axis_index("c")
        row0 = pl.multiple_of((wid // n_col_parts) * RG, RG)
        col_base = pl.multiple_of((wid % n_col_parts) * cols_per_tile, CHUNK_C)
        in_bufs, out_bufs = (in0_v, in1_v), (out0_v, out1_v)
        in_sems, out_sems = (isem0, isem1), (osem0, osem1)

        def in_copy(ci):
            c0 = pl.multiple_of(col_base + ci * CHUNK_C, CHUNK_C)
            b = ci & 1
            return pltpu.make_async_copy(
                t_hbm.at[pl.ds(row0, RG), pl.ds(c0, CHUNK_C)],
                in_bufs[b], in_sems[b])

        def out_copy(ci):
            c0 = pl.multiple_of(col_base + ci * CHUNK_C, CHUNK_C)
            b = ci & 1
            return pltpu.make_async_copy(
                out_bufs[b],
                out_hbm.at[pl.ds(row0, RG), pl.ds(c0, CHUNK_C)], out_sems[b])

        in_copy(0).start()
        if n_chunks > 1:
            in_copy(1).start()

        pltpu.sync_copy(pat_hbm, pat_v.at[pl.ds(0, num_ops)])
        zeros = jnp.zeros((L,), jnp.int32)
        for j in range(n_pat_vregs):
            lut_v[pl.ds(j * L, L)] = zeros
        for j in reversed(range(n_pat_vregs)):
            pats = pat_v[pl.ds(j * L, L)]
            vals = lax.iota(jnp.int32, L) + (j * L)
            plsc.store_scatter(lut_v, [pats], vals)

        for ci in range(n_chunks):
            b = ci & 1
            in_copy(ci).wait()
            if ci >= 2:
                out_copy(ci - 2).wait()

            for r in range(RG):
                def vec_body(g, c2, b=b, r=r):
                    for u in range(UNROLL):
                        o = pl.multiple_of(g * (L * UNROLL) + u * L, L)
                        idx = in_bufs[b][r, pl.ds(o, L)]
                        out_bufs[b][r, pl.ds(o, L)] = plsc.load_gather(
                            lut_v, [idx])
                    return c2

                lax.fori_loop(0, CHUNK_C // (L * UNROLL), vec_body, 0)

            out_copy(ci).start()
            if ci + 2 < n_chunks:
                in_copy(ci + 2).start()

        for ci in range(max(0, n_chunks - 2), n_chunks):
            out_copy(ci).wait()

    return lookup


def kernel(tensor, op_patterns):
    nrows, ncols = tensor.shape
    lookup = _build_lookup(nrows, ncols, op_patterns.shape[0])
    return lookup(tensor, op_patterns)


# speedup vs baseline: 1.0485x; 1.0023x over previous
import functools

import jax
import jax.numpy as jnp
from jax import lax
from jax.experimental import pallas as pl
from jax.experimental.pallas import tpu as pltpu
from jax.experimental.pallas import tpu_sc as plsc

L = 16
NC = 2
NS = 16
NW = NC * NS
RG = 8
CHUNK_C = 2048
UNROLL = 16


@functools.cache
def _build_lookup(nrows: int, ncols: int, num_ops: int):
    n_row_groups = nrows // RG
    n_col_parts = NW // n_row_groups
    cols_per_tile = ncols // n_col_parts
    n_chunks = cols_per_tile // CHUNK_C
    n_pat_vregs = num_ops // L

    mesh = plsc.VectorSubcoreMesh(core_axis_name="c", subcore_axis_name="s")

    @functools.partial(
        pl.kernel,
        mesh=mesh,
        compiler_params=pltpu.CompilerParams(
            needs_layout_passes=False,
            disable_bounds_checks=True,
            skip_device_barrier=True,
        ),
        out_type=jax.ShapeDtypeStruct((nrows, ncols), jnp.int32),
        scratch_types=[
            pltpu.VMEM((max(num_ops, 128),), jnp.int32),
            pltpu.VMEM((max(num_ops, 128),), jnp.int32),
            pltpu.VMEM((RG, CHUNK_C), jnp.int32),
            pltpu.VMEM((RG, CHUNK_C), jnp.int32),
            pltpu.VMEM((RG, CHUNK_C), jnp.int32),
            pltpu.VMEM((RG, CHUNK_C), jnp.int32),
            pltpu.SemaphoreType.DMA,
            pltpu.SemaphoreType.DMA,
            pltpu.SemaphoreType.DMA,
            pltpu.SemaphoreType.DMA,
        ],
    )
    def lookup(t_hbm, pat_hbm, out_hbm, pat_v, lut_v,
               in0_v, in1_v, out0_v, out1_v, isem0, isem1, osem0, osem1):
        wid = lax.axis_index("s") * NC + lax.axis_index("c")
        row0 = pl.multiple_of((wid // n_col_parts) * RG, RG)
        col_base = pl.multiple_of((wid % n_col_parts) * cols_per_tile, CHUNK_C)
        in_bufs, out_bufs = (in0_v, in1_v), (out0_v, out1_v)
        in_sems, out_sems = (isem0, isem1), (osem0, osem1)

        def in_copy(ci):
            c0 = pl.multiple_of(col_base + ci * CHUNK_C, CHUNK_C)
            b = ci & 1
            return pltpu.make_async_copy(
                t_hbm.at[pl.ds(row0, RG), pl.ds(c0, CHUNK_C)],
                in_bufs[b], in_sems[b])

        def out_copy(ci):
            c0 = pl.multiple_of(col_base + ci * CHUNK_C, CHUNK_C)
            b = ci & 1
            return pltpu.make_async_copy(
                out_bufs[b],
                out_hbm.at[pl.ds(row0, RG), pl.ds(c0, CHUNK_C)], out_sems[b])

        in_copy(0).start()
        if n_chunks > 1:
            in_copy(1).start()

        pltpu.sync_copy(pat_hbm, pat_v.at[pl.ds(0, num_ops)])
        zeros = jnp.zeros((L,), jnp.int32)
        for j in range(n_pat_vregs):
            lut_v[pl.ds(j * L, L)] = zeros
        for j in reversed(range(n_pat_vregs)):
            pats = pat_v[pl.ds(j * L, L)]
            vals = lax.iota(jnp.int32, L) + (j * L)
            plsc.store_scatter(lut_v, [pats], vals)

        for ci in range(n_chunks):
            b = ci & 1
            in_copy(ci).wait()
            if ci >= 2:
                out_copy(ci - 2).wait()

            for r in range(RG):
                def vec_body(g, c2, b=b, r=r):
                    for u in range(UNROLL):
                        o = pl.multiple_of(g * (L * UNROLL) + u * L, L)
                        idx = in_bufs[b][r, pl.ds(o, L)]
                        out_bufs[b][r, pl.ds(o, L)] = plsc.load_gather(
                            lut_v, [idx])
                    return c2

                lax.fori_loop(0, CHUNK_C // (L * UNROLL), vec_body, 0)

            out_copy(ci).start()
            if ci + 2 < n_chunks:
                in_copy(ci + 2).start()

        for ci in range(max(0, n_chunks - 2), n_chunks):
            out_copy(ci).wait()

    return lookup


def kernel(tensor, op_patterns):
    nrows, ncols = tensor.shape
    lookup = _build_lookup(nrows, ncols, op_patterns.shape[0])
    return lookup(tensor, op_patterns)


# speedup vs baseline: 1.0901x; 1.0396x over previous
import functools

import jax
import jax.numpy as jnp
from jax import lax
from jax.experimental import pallas as pl
from jax.experimental.pallas import tpu as pltpu
from jax.experimental.pallas import tpu_sc as plsc

L = 16
NC = 2
NS = 16
NW = NC * NS
RG = 8
CHUNK_C = 2048
UNROLL = 8


@functools.cache
def _build_lookup(nrows: int, ncols: int, num_ops: int):
    n_row_groups = nrows // RG
    n_col_parts = NW // n_row_groups
    cols_per_tile = ncols // n_col_parts
    n_chunks = cols_per_tile // CHUNK_C
    n_pat_vregs = num_ops // L

    mesh = plsc.VectorSubcoreMesh(core_axis_name="c", subcore_axis_name="s")

    @functools.partial(
        pl.kernel,
        mesh=mesh,
        compiler_params=pltpu.CompilerParams(
            needs_layout_passes=False,
            disable_bounds_checks=True,
            skip_device_barrier=True,
        ),
        out_type=jax.ShapeDtypeStruct((nrows, ncols), jnp.int32),
        scratch_types=[
            pltpu.VMEM((max(num_ops, 128),), jnp.int32),
            pltpu.VMEM((num_ops * L,), jnp.int32),
            pltpu.VMEM((RG, CHUNK_C), jnp.int32),
            pltpu.VMEM((RG, CHUNK_C), jnp.int32),
            pltpu.VMEM((RG, CHUNK_C), jnp.int32),
            pltpu.VMEM((RG, CHUNK_C), jnp.int32),
            pltpu.SemaphoreType.DMA,
            pltpu.SemaphoreType.DMA,
            pltpu.SemaphoreType.DMA,
            pltpu.SemaphoreType.DMA,
        ],
    )
    def lookup(t_hbm, pat_hbm, out_hbm, pat_v, lut_v,
               in0_v, in1_v, out0_v, out1_v, isem0, isem1, osem0, osem1):
        wid = lax.axis_index("s") * NC + lax.axis_index("c")
        row0 = pl.multiple_of((wid // n_col_parts) * RG, RG)
        col_base = pl.multiple_of((wid % n_col_parts) * cols_per_tile, CHUNK_C)
        in_bufs, out_bufs = (in0_v, in1_v), (out0_v, out1_v)
        in_sems, out_sems = (isem0, isem1), (osem0, osem1)

        def in_copy(ci):
            c0 = pl.multiple_of(col_base + ci * CHUNK_C, CHUNK_C)
            b = ci & 1
            return pltpu.make_async_copy(
                t_hbm.at[pl.ds(row0, RG), pl.ds(c0, CHUNK_C)],
                in_bufs[b], in_sems[b])

        def out_copy(ci):
            c0 = pl.multiple_of(col_base + ci * CHUNK_C, CHUNK_C)
            b = ci & 1
            return pltpu.make_async_copy(
                out_bufs[b],
                out_hbm.at[pl.ds(row0, RG), pl.ds(c0, CHUNK_C)], out_sems[b])

        in_copy(0).start()
        if n_chunks > 1:
            in_copy(1).start()

        pltpu.sync_copy(pat_hbm, pat_v.at[pl.ds(0, num_ops)])
        zeros = jnp.zeros((L,), jnp.int32)
        lane = lax.iota(jnp.int32, L)
        for v in range(num_ops):
            lut_v[pl.ds(v * L, L)] = zeros
        for j in reversed(range(n_pat_vregs)):
            pats = pat_v[pl.ds(j * L, L)]
            vals = lax.iota(jnp.int32, L) + (j * L)
            scaled = jnp.left_shift(pats, 4)
            for l in range(L):
                plsc.store_scatter(lut_v, [scaled + l], vals)

        for ci in range(n_chunks):
            b = ci & 1
            in_copy(ci).wait()
            if ci >= 2:
                out_copy(ci - 2).wait()

            for r in range(RG):
                def vec_body(g, c2, b=b, r=r):
                    for u in range(UNROLL):
                        o = pl.multiple_of(g * (L * UNROLL) + u * L, L)
                        idx = in_bufs[b][r, pl.ds(o, L)]
                        addr = jnp.bitwise_or(jnp.left_shift(idx, 4), lane)
                        out_bufs[b][r, pl.ds(o, L)] = plsc.load_gather(
                            lut_v, [addr])
                    return c2

                lax.fori_loop(0, CHUNK_C // (L * UNROLL), vec_body, 0)

            out_copy(ci).start()
            if ci + 2 < n_chunks:
                in_copy(ci + 2).start()

        for ci in range(max(0, n_chunks - 2), n_chunks):
            out_copy(ci).wait()

    return lookup


def kernel(tensor, op_patterns):
    nrows, ncols = tensor.shape
    lookup = _build_lookup(nrows, ncols, op_patterns.shape[0])
    return lookup(tensor, op_patterns)


# speedup vs baseline: 1.5176x; 1.3922x over previous
import functools

import jax
import jax.numpy as jnp
from jax import lax
from jax.experimental import pallas as pl
from jax.experimental.pallas import tpu as pltpu
from jax.experimental.pallas import tpu_sc as plsc

L = 16
NC = 2
NS = 16
NW = NC * NS
RG = 8
CHUNK_C = 2048
UNROLL = 8


@functools.cache
def _build_lookup(nrows: int, ncols: int, num_ops: int):
    n_row_groups = nrows // RG
    n_col_parts = NW // n_row_groups
    cols_per_tile = ncols // n_col_parts
    n_chunks = cols_per_tile // CHUNK_C
    n_pat_vregs = num_ops // L

    mesh = plsc.VectorSubcoreMesh(core_axis_name="c", subcore_axis_name="s")

    @functools.partial(
        pl.kernel,
        mesh=mesh,
        compiler_params=pltpu.CompilerParams(
            needs_layout_passes=False,
            disable_bounds_checks=True,
            skip_device_barrier=True,
        ),
        out_type=jax.ShapeDtypeStruct((nrows, ncols), jnp.int32),
        scratch_types=[
            pltpu.VMEM((max(num_ops, 128),), jnp.int32),
            pltpu.VMEM((num_ops * L,), jnp.int32),
            pltpu.VMEM((RG, CHUNK_C), jnp.int32),
            pltpu.VMEM((RG, CHUNK_C), jnp.int32),
            pltpu.VMEM((RG, CHUNK_C), jnp.int32),
            pltpu.VMEM((RG, CHUNK_C), jnp.int32),
            pltpu.SemaphoreType.DMA,
            pltpu.SemaphoreType.DMA,
            pltpu.SemaphoreType.DMA,
            pltpu.SemaphoreType.DMA,
        ],
    )
    def lookup(t_hbm, pat_hbm, out_hbm, pat_v, lut_v,
               in0_v, in1_v, out0_v, out1_v, isem0, isem1, osem0, osem1):
        wid = lax.axis_index("s") * NC + lax.axis_index("c")
        row0 = pl.multiple_of((wid // n_col_parts) * RG, RG)
        col_base = pl.multiple_of((wid % n_col_parts) * cols_per_tile, CHUNK_C)
        in_bufs, out_bufs = (in0_v, in1_v), (out0_v, out1_v)
        in_sems, out_sems = (isem0, isem1), (osem0, osem1)

        def in_copy(ci):
            c0 = pl.multiple_of(col_base + ci * CHUNK_C, CHUNK_C)
            b = ci & 1
            return pltpu.make_async_copy(
                t_hbm.at[pl.ds(row0, RG), pl.ds(c0, CHUNK_C)],
                in_bufs[b], in_sems[b])

        def out_copy(ci):
            c0 = pl.multiple_of(col_base + ci * CHUNK_C, CHUNK_C)
            b = ci & 1
            return pltpu.make_async_copy(
                out_bufs[b],
                out_hbm.at[pl.ds(row0, RG), pl.ds(c0, CHUNK_C)], out_sems[b])

        in_copy(0).start()
        if n_chunks > 1:
            in_copy(1).start()

        pltpu.sync_copy(pat_hbm, pat_v.at[pl.ds(0, num_ops)])
        zeros = jnp.zeros((L,), jnp.int32)
        lane = lax.iota(jnp.int32, L)
        for v in range(num_ops):
            lut_v[pl.ds(v * L, L)] = zeros
        for j in reversed(range(n_pat_vregs)):
            pats = pat_v[pl.ds(j * L, L)]
            vals = lax.iota(jnp.int32, L) + (j * L)
            scaled = jnp.left_shift(pats, 4)
            for l in range(L):
                plsc.store_scatter(lut_v, [scaled + l], vals)

        for ci in range(n_chunks):
            b = ci & 1
            in_copy(ci).wait()
            if ci >= 2:
                out_copy(ci - 2).wait()

            for r in range(RG):
                @plsc.parallel_loop(0, CHUNK_C, step=L, unroll=UNROLL)
                def vec_body(o, b=b, r=r):
                    o = pl.multiple_of(o, L)
                    idx = in_bufs[b][r, pl.ds(o, L)]
                    addr = jnp.bitwise_or(jnp.left_shift(idx, 4), lane)
                    out_bufs[b][r, pl.ds(o, L)] = plsc.load_gather(
                        lut_v, [addr])

            out_copy(ci).start()
            if ci + 2 < n_chunks:
                in_copy(ci + 2).start()

        for ci in range(max(0, n_chunks - 2), n_chunks):
            out_copy(ci).wait()

    return lookup


def kernel(tensor, op_patterns):
    nrows, ncols = tensor.shape
    lookup = _build_lookup(nrows, ncols, op_patterns.shape[0])
    return lookup(tensor, op_patterns)
